# baseline (speedup 1.0000x reference)
import functools

import jax
import jax.numpy as jnp
from jax import lax
from jax.experimental import pallas as pl
from jax.experimental.pallas import tpu as pltpu
from jax.experimental.pallas import tpu_sc as plsc

NUM_EXPERTS = 8
TOP_K = 2
D_MODEL = 1024
D_FF = 4096
N_TOKENS = 2048

NPAIR = N_TOKENS * TOP_K
BLK = 256
NBLK = 24
NSLOT = NBLK * BLK
NWORKER = 32
FF_CHUNK = 2048
FF_STEPS = D_FF // FF_CHUNK

_MESH = plsc.VectorSubcoreMesh(core_axis_name="c", subcore_axis_name="s")


def _gate_body(x_ref, wg_ref, id1_ref, id2_ref, w1_ref, w2_ref):
    x = x_ref[...]
    wg = wg_ref[...]
    logits = jnp.dot(x, wg, preferred_element_type=jnp.float32)
    iota = jax.lax.broadcasted_iota(jnp.int32, logits.shape, 1)
    m1 = jnp.max(logits, axis=1, keepdims=True)
    am1 = jnp.min(jnp.where(logits == m1, iota, NUM_EXPERTS), axis=1,
                  keepdims=True)
    masked = jnp.where(iota == am1, -jnp.inf, logits)
    m2 = jnp.max(masked, axis=1, keepdims=True)
    am2 = jnp.min(jnp.where(masked == m2, iota, NUM_EXPERTS), axis=1,
                  keepdims=True)
    z = jnp.exp(m2 - m1)
    id1_ref[...] = am1
    id2_ref[...] = am2
    w1_ref[...] = 1.0 / (1.0 + z)
    w2_ref[...] = z / (1.0 + z)


def _gate(inputs, W_gate):
    return pl.pallas_call(
        _gate_body,
        out_shape=(
            jax.ShapeDtypeStruct((N_TOKENS, 1), jnp.int32),
            jax.ShapeDtypeStruct((N_TOKENS, 1), jnp.int32),
            jax.ShapeDtypeStruct((N_TOKENS, 1), jnp.float32),
            jax.ShapeDtypeStruct((N_TOKENS, 1), jnp.float32),
        ),
    )(inputs, W_gate)


def _take(vec, idx):
    idxv = jnp.full((16, 1), idx, jnp.int32)
    dnums = lax.GatherDimensionNumbers(
        offset_dims=(), collapsed_slice_dims=(0,), start_index_map=(0,))
    return lax.gather(vec, idxv, dnums, slice_sizes=(1,),
                      mode=lax.GatherScatterMode.PROMISE_IN_BOUNDS)


@functools.partial(
    pl.kernel, mesh=_MESH,
    compiler_params=pltpu.CompilerParams(needs_layout_passes=False),
    out_type=(
        jax.ShapeDtypeStruct((NSLOT,), jnp.int32),
        jax.ShapeDtypeStruct((N_TOKENS,), jnp.int32),
        jax.ShapeDtypeStruct((N_TOKENS,), jnp.int32),
        jax.ShapeDtypeStruct((NWORKER,), jnp.int32),
        jax.ShapeDtypeStruct((NWORKER,), jnp.int32),
    ),
    scratch_types=[
        pltpu.VMEM((NPAIR,), jnp.int32),
        pltpu.VMEM((NSLOT,), jnp.int32),
        pltpu.VMEM((N_TOKENS,), jnp.int32),
        pltpu.VMEM((N_TOKENS,), jnp.int32),
        pltpu.VMEM((16,), jnp.int32),
        pltpu.VMEM((NWORKER,), jnp.int32),
        pltpu.VMEM((NWORKER,), jnp.int32),
    ],
)
def _route(ids_hbm, stok_hbm, pos0_hbm, pos1_hbm, be_hbm, bv_hbm,
           ids_v, stok_v, pos0_v, pos1_v, counts_v, be_v, bv_v):
    cid = lax.axis_index("c")
    sid = lax.axis_index("s")

    @pl.when(jnp.logical_and(cid == 0, sid == 0))
    def _():
        lanes = lax.iota(jnp.int32, 16)
        pltpu.sync_copy(ids_hbm, ids_v)
        counts_v[...] = jnp.zeros((16,), jnp.int32)

        def hist_body(i, carry):
            ids = ids_v[pl.ds(i * 16, 16)]
            plsc.addupdate_scatter(counts_v, [ids],
                                   jnp.ones((16,), jnp.int32))
            return carry

        lax.fori_loop(0, NPAIR // 16, hist_body, 0)

        counts = counts_v[...]
        aligned = ((counts + (BLK - 1)) // BLK) * BLK
        incl = plsc.cumsum(aligned)
        excl = incl - aligned
        total_blk = _take(incl, NUM_EXPERTS - 1) // BLK
        last_e = jnp.max(jnp.where(counts > 0, lanes, 0))

        for j in range(NWORKER // 16):
            bidx = lanes + 16 * j
            acc = jnp.zeros((16,), jnp.int32)
            for e in range(NUM_EXPERTS):
                acc += (bidx * BLK >= _take(excl, e)).astype(jnp.int32)
            eb = jnp.minimum(acc - 1, NUM_EXPERTS - 1)
            valid = (bidx < total_blk).astype(jnp.int32)
            eb = jnp.where(valid == 1, eb, last_e)
            be_v[pl.ds(16 * j, 16)] = eb
            bv_v[pl.ds(16 * j, 16)] = valid

        def zero_body(i, carry):
            stok_v[pl.ds(i * 16, 16)] = jnp.zeros((16,), jnp.int32)
            return carry

        lax.fori_loop(0, NSLOT // 16, zero_body, 0)

        def pos_body(i, counters):
            ids = ids_v[pl.ds(i * 16, 16)]
            p = jnp.zeros((16,), jnp.int32)
            upd = jnp.zeros((16,), jnp.int32)
            for e in range(NUM_EXPERTS):
                m = ids == e
                csum = plsc.cumsum(m.astype(jnp.int32))
                p = jnp.where(m, _take(counters, e) + csum - 1, p)
                upd += jnp.where(lanes == e, _take(csum, 15), 0)
            pairidx = lanes + i * 16
            tok = pairidx // 2
            k = pairidx % 2
            plsc.store_scatter(stok_v, [p], tok)
            plsc.store_scatter(pos0_v, [tok], p, mask=(k == 0))
            plsc.store_scatter(pos1_v, [tok], p, mask=(k == 1))
            return counters + upd

        lax.fori_loop(0, NPAIR // 16, pos_body, excl)

        pltpu.sync_copy(stok_v, stok_hbm)
        pltpu.sync_copy(pos0_v, pos0_hbm)
        pltpu.sync_copy(pos1_v, pos1_hbm)
        pltpu.sync_copy(be_v, be_hbm)
        pltpu.sync_copy(bv_v, bv_hbm)


GAT_PER_W = NSLOT // NWORKER
GAT_CHUNK = 24
GAT_NCHUNK = GAT_PER_W // GAT_CHUNK
GAT_NBUF = 4


@functools.partial(
    pl.kernel, mesh=_MESH,
    compiler_params=pltpu.CompilerParams(needs_layout_passes=False),
    out_type=jax.ShapeDtypeStruct((NSLOT, D_MODEL), jnp.float32),
    scratch_types=(
        [pltpu.VMEM((GAT_PER_W,), jnp.int32)]
        + [pltpu.VMEM((GAT_CHUNK, D_MODEL), jnp.float32)] * GAT_NBUF
        + [pltpu.SemaphoreType.DMA] * GAT_NBUF
        + [pltpu.SemaphoreType.DMA] * GAT_NBUF
    ),
)
def _gather(x_hbm, stok_hbm, xs_hbm, idx_v, *bufs_sems):
    bufs = bufs_sems[:GAT_NBUF]
    gsem = bufs_sems[GAT_NBUF:2 * GAT_NBUF]
    wsem = bufs_sems[2 * GAT_NBUF:]
    wid = lax.axis_index("s") * 2 + lax.axis_index("c")
    base = wid * GAT_PER_W
    pltpu.sync_copy(stok_hbm.at[pl.ds(base, GAT_PER_W)], idx_v)
    gcp = [None] * GAT_NCHUNK
    wcp = [None] * GAT_NCHUNK
    for t in range(GAT_NCHUNK + 2):
        if t < GAT_NCHUNK:
            if t >= GAT_NBUF:
                wcp[t - GAT_NBUF].wait()
            gcp[t] = pltpu.async_copy(
                x_hbm.at[idx_v.at[pl.ds(t * GAT_CHUNK, GAT_CHUNK)]],
                bufs[t % GAT_NBUF], gsem[t % GAT_NBUF])
        if t >= 2:
            c = t - 2
            gcp[c].wait()
            wcp[c] = pltpu.async_copy(
                bufs[c % GAT_NBUF],
                xs_hbm.at[pl.ds(base + c * GAT_CHUNK, GAT_CHUNK)],
                wsem[c % GAT_NBUF])
    wcp[GAT_NCHUNK - 2].wait()
    wcp[GAT_NCHUNK - 1].wait()


def _gmm_body(be_ref, bv_ref, x_ref, w1_ref, w2_ref, out_ref):
    @pl.when(bv_ref[pl.program_id(0)] == 1)
    def _():
        xb = x_ref[...].astype(jnp.bfloat16)
        h = jnp.dot(xb, w1_ref[0], preferred_element_type=jnp.float32)
        hb = jax.nn.gelu(h).astype(jnp.bfloat16)
        out_ref[...] = jnp.dot(hb, w2_ref[0], preferred_element_type=jnp.float32)


def _gmm(be, bv, xs, W1b, W2b):
    grid_spec = pltpu.PrefetchScalarGridSpec(
        num_scalar_prefetch=2,
        grid=(NBLK,),
        in_specs=[
            pl.BlockSpec((BLK, D_MODEL),
                         lambda b, be, bv: (jnp.where(bv[b] == 1, b, 0), 0)),
            pl.BlockSpec((1, D_MODEL, D_FF), lambda b, be, bv: (be[b], 0, 0)),
            pl.BlockSpec((1, D_FF, D_MODEL), lambda b, be, bv: (be[b], 0, 0)),
        ],
        out_specs=pl.BlockSpec((BLK, D_MODEL), lambda b, be, bv: (b, 0)),
    )
    return pl.pallas_call(
        _gmm_body,
        grid_spec=grid_spec,
        out_shape=jax.ShapeDtypeStruct((NSLOT, D_MODEL), jnp.float32),
    )(be, bv, xs, W1b, W2b)


CMB_PER_W = N_TOKENS // NWORKER
CMB_CHUNK = 16


CMB_NCHUNK = CMB_PER_W // CMB_CHUNK


@functools.partial(
    pl.kernel, mesh=_MESH,
    compiler_params=pltpu.CompilerParams(needs_layout_passes=False),
    out_type=jax.ShapeDtypeStruct((N_TOKENS, D_MODEL), jnp.float32),
    scratch_types=(
        [pltpu.VMEM((CMB_PER_W,), jnp.int32)] * 2
        + [pltpu.VMEM((CMB_PER_W,), jnp.float32)] * 2
        + [pltpu.VMEM((CMB_CHUNK, D_MODEL), jnp.float32)] * 6
        + [pltpu.SemaphoreType.DMA] * 6
    ),
)
def _combine(y_hbm, pos0_hbm, pos1_hbm, w0_hbm, w1_hbm, out_hbm,
             p0_v, p1_v, w0_v, w1_v, *bufs_sems):
    r0b = bufs_sems[0:2]
    r1b = bufs_sems[2:4]
    obb = bufs_sems[4:6]
    gs0 = bufs_sems[6:8]
    gs1 = bufs_sems[8:10]
    ws = bufs_sems[10:12]
    wid = lax.axis_index("s") * 2 + lax.axis_index("c")
    tbase = wid * CMB_PER_W
    pltpu.sync_copy(pos0_hbm.at[pl.ds(tbase, CMB_PER_W)], p0_v)
    pltpu.sync_copy(pos1_hbm.at[pl.ds(tbase, CMB_PER_W)], p1_v)
    pltpu.sync_copy(w0_hbm.at[pl.ds(tbase, CMB_PER_W)], w0_v)
    pltpu.sync_copy(w1_hbm.at[pl.ds(tbase, CMB_PER_W)], w1_v)

    g0cp = [None] * CMB_NCHUNK
    g1cp = [None] * CMB_NCHUNK
    wcp = [None] * CMB_NCHUNK

    def issue(c):
        g0cp[c] = pltpu.async_copy(
            y_hbm.at[p0_v.at[pl.ds(c * CMB_CHUNK, CMB_CHUNK)]],
            r0b[c % 2], gs0[c % 2])
        g1cp[c] = pltpu.async_copy(
            y_hbm.at[p1_v.at[pl.ds(c * CMB_CHUNK, CMB_CHUNK)]],
            r1b[c % 2], gs1[c % 2])

    issue(0)
    for c in range(CMB_NCHUNK):
        if c + 1 < CMB_NCHUNK:
            issue(c + 1)
        g0cp[c].wait()
        g1cp[c].wait()
        if c >= 2:
            wcp[c - 2].wait()
        wa = w0_v[pl.ds(c * CMB_CHUNK, CMB_CHUNK)]
        wb = w1_v[pl.ds(c * CMB_CHUNK, CMB_CHUNK)]
        r0c = r0b[c % 2]
        r1c = r1b[c % 2]
        obc = obb[c % 2]
        for j in range(CMB_CHUNK):
            s0 = _take(wa, j)
            s1 = _take(wb, j)

            def vbody(v, carry, j=j, s0=s0, s1=s1, r0c=r0c, r1c=r1c, obc=obc):
                for u in range(4):
                    off = v * 64 + u * 16
                    obc[j, pl.ds(off, 16)] = (
                        s0 * r0c[j, pl.ds(off, 16)]
                        + s1 * r1c[j, pl.ds(off, 16)])
                return carry

            lax.fori_loop(0, D_MODEL // 64, vbody, 0)
        wcp[c] = pltpu.async_copy(
            obc, out_hbm.at[pl.ds(tbase + c * CMB_CHUNK, CMB_CHUNK)],
            ws[c % 2])
    wcp[CMB_NCHUNK - 2].wait()
    wcp[CMB_NCHUNK - 1].wait()


def kernel(inputs, W_gate, W1, W2):
    W1b = W1.astype(jnp.bfloat16)
    W2b = W2.astype(jnp.bfloat16)
    id1, id2, w1, w2 = _gate(inputs, W_gate)
    ids_flat = jnp.concatenate([id1, id2], axis=1).reshape(NPAIR)
    stok, pos0, pos1, be, bv = _route(ids_flat)
    xs = _gather(inputs, stok)
    ys = _gmm(be, bv, xs, W1b, W2b)
    return _combine(ys, pos0, pos1, w1.reshape(-1), w2.reshape(-1))


# speedup vs baseline: 1.1520x; 1.1520x over previous
import functools

import jax
import jax.numpy as jnp
from jax import lax
from jax.experimental import pallas as pl
from jax.experimental.pallas import tpu as pltpu
from jax.experimental.pallas import tpu_sc as plsc

NUM_EXPERTS = 8
TOP_K = 2
D_MODEL = 1024
D_FF = 4096
N_TOKENS = 2048

NPAIR = N_TOKENS * TOP_K
BLK = 256
NBLK = 24
NSLOT = NBLK * BLK
NWORKER = 32
FF_CHUNK = 2048
FF_STEPS = D_FF // FF_CHUNK

_MESH = plsc.VectorSubcoreMesh(core_axis_name="c", subcore_axis_name="s")


def _gate_body(x_ref, wg_ref, id1_ref, id2_ref, w1_ref, w2_ref):
    x = x_ref[...]
    wg = wg_ref[...]
    logits = jnp.dot(x, wg, preferred_element_type=jnp.float32)
    iota = jax.lax.broadcasted_iota(jnp.int32, logits.shape, 1)
    m1 = jnp.max(logits, axis=1, keepdims=True)
    am1 = jnp.min(jnp.where(logits == m1, iota, NUM_EXPERTS), axis=1,
                  keepdims=True)
    masked = jnp.where(iota == am1, -jnp.inf, logits)
    m2 = jnp.max(masked, axis=1, keepdims=True)
    am2 = jnp.min(jnp.where(masked == m2, iota, NUM_EXPERTS), axis=1,
                  keepdims=True)
    z = jnp.exp(m2 - m1)
    id1_ref[...] = am1
    id2_ref[...] = am2
    w1_ref[...] = 1.0 / (1.0 + z)
    w2_ref[...] = z / (1.0 + z)


def _gate(inputs, W_gate):
    return pl.pallas_call(
        _gate_body,
        out_shape=(
            jax.ShapeDtypeStruct((N_TOKENS, 1), jnp.int32),
            jax.ShapeDtypeStruct((N_TOKENS, 1), jnp.int32),
            jax.ShapeDtypeStruct((N_TOKENS, 1), jnp.float32),
            jax.ShapeDtypeStruct((N_TOKENS, 1), jnp.float32),
        ),
    )(inputs, W_gate)


def _take(vec, idx):
    idxv = jnp.full((16, 1), idx, jnp.int32)
    dnums = lax.GatherDimensionNumbers(
        offset_dims=(), collapsed_slice_dims=(0,), start_index_map=(0,))
    return lax.gather(vec, idxv, dnums, slice_sizes=(1,),
                      mode=lax.GatherScatterMode.PROMISE_IN_BOUNDS)


@functools.partial(
    pl.kernel, mesh=_MESH,
    compiler_params=pltpu.CompilerParams(needs_layout_passes=False),
    out_type=(
        jax.ShapeDtypeStruct((NSLOT,), jnp.int32),
        jax.ShapeDtypeStruct((N_TOKENS,), jnp.int32),
        jax.ShapeDtypeStruct((N_TOKENS,), jnp.int32),
        jax.ShapeDtypeStruct((NWORKER,), jnp.int32),
        jax.ShapeDtypeStruct((NWORKER,), jnp.int32),
    ),
    scratch_types=[
        pltpu.VMEM((NPAIR,), jnp.int32),
        pltpu.VMEM((NSLOT,), jnp.int32),
        pltpu.VMEM((N_TOKENS,), jnp.int32),
        pltpu.VMEM((N_TOKENS,), jnp.int32),
        pltpu.VMEM((16,), jnp.int32),
        pltpu.VMEM((NWORKER,), jnp.int32),
        pltpu.VMEM((NWORKER,), jnp.int32),
    ],
)
def _route(ids_hbm, stok_hbm, pos0_hbm, pos1_hbm, be_hbm, bv_hbm,
           ids_v, stok_v, pos0_v, pos1_v, counts_v, be_v, bv_v):
    cid = lax.axis_index("c")
    sid = lax.axis_index("s")

    @pl.when(jnp.logical_and(cid == 0, sid == 0))
    def _():
        lanes = lax.iota(jnp.int32, 16)
        pltpu.sync_copy(ids_hbm, ids_v)
        counts_v[...] = jnp.zeros((16,), jnp.int32)

        def hist_body(i, carry):
            ids = ids_v[pl.ds(i * 16, 16)]
            plsc.addupdate_scatter(counts_v, [ids],
                                   jnp.ones((16,), jnp.int32))
            return carry

        lax.fori_loop(0, NPAIR // 16, hist_body, 0)

        counts = counts_v[...]
        aligned = ((counts + (BLK - 1)) // BLK) * BLK
        incl = plsc.cumsum(aligned)
        excl = incl - aligned
        total_blk = _take(incl, NUM_EXPERTS - 1) // BLK
        last_e = jnp.max(jnp.where(counts > 0, lanes, 0))

        for j in range(NWORKER // 16):
            bidx = lanes + 16 * j
            acc = jnp.zeros((16,), jnp.int32)
            for e in range(NUM_EXPERTS):
                acc += (bidx * BLK >= _take(excl, e)).astype(jnp.int32)
            eb = jnp.minimum(acc - 1, NUM_EXPERTS - 1)
            valid = (bidx < total_blk).astype(jnp.int32)
            eb = jnp.where(valid == 1, eb, last_e)
            be_v[pl.ds(16 * j, 16)] = eb
            bv_v[pl.ds(16 * j, 16)] = valid

        def zero_body(i, carry):
            stok_v[pl.ds(i * 16, 16)] = jnp.zeros((16,), jnp.int32)
            return carry

        lax.fori_loop(0, NSLOT // 16, zero_body, 0)

        def pos_body(i, counters):
            ids = ids_v[pl.ds(i * 16, 16)]
            p = jnp.zeros((16,), jnp.int32)
            upd = jnp.zeros((16,), jnp.int32)
            for e in range(NUM_EXPERTS):
                m = ids == e
                csum = plsc.cumsum(m.astype(jnp.int32))
                p = jnp.where(m, _take(counters, e) + csum - 1, p)
                upd += jnp.where(lanes == e, _take(csum, 15), 0)
            pairidx = lanes + i * 16
            tok = pairidx // 2
            k = pairidx % 2
            plsc.store_scatter(stok_v, [p], tok)
            plsc.store_scatter(pos0_v, [tok], p, mask=(k == 0))
            plsc.store_scatter(pos1_v, [tok], p, mask=(k == 1))
            return counters + upd

        lax.fori_loop(0, NPAIR // 16, pos_body, excl)

        pltpu.sync_copy(stok_v, stok_hbm)
        pltpu.sync_copy(pos0_v, pos0_hbm)
        pltpu.sync_copy(pos1_v, pos1_hbm)
        pltpu.sync_copy(be_v, be_hbm)
        pltpu.sync_copy(bv_v, bv_hbm)


GAT_PER_W = NSLOT // NWORKER
GAT_CHUNK = 48
GAT_NCHUNK = GAT_PER_W // GAT_CHUNK


@functools.partial(
    pl.kernel, mesh=_MESH,
    compiler_params=pltpu.CompilerParams(needs_layout_passes=False),
    out_type=jax.ShapeDtypeStruct((NSLOT, D_MODEL), jnp.float32),
    scratch_types=(
        [pltpu.VMEM((GAT_PER_W,), jnp.int32)]
        + [pltpu.VMEM((GAT_CHUNK, D_MODEL), jnp.float32)] * 2
        + [pltpu.SemaphoreType.DMA] * 4
    ),
)
def _gather(x_hbm, stok_hbm, xs_hbm, idx_v, buf0, buf1, gs0, gs1, ws0, ws1):
    bufs = (buf0, buf1)
    gsem = (gs0, gs1)
    wsem = (ws0, ws1)
    wid = lax.axis_index("s") * 2 + lax.axis_index("c")
    base = wid * GAT_PER_W
    pltpu.sync_copy(stok_hbm.at[pl.ds(base, GAT_PER_W)], idx_v)
    gcp = [None] * GAT_NCHUNK
    wcp = [None] * GAT_NCHUNK

    def issue(c):
        gcp[c] = pltpu.async_copy(
            x_hbm.at[idx_v.at[pl.ds(c * GAT_CHUNK, GAT_CHUNK)]],
            bufs[c % 2], gsem[c % 2])

    issue(0)
    for c in range(GAT_NCHUNK):
        if c + 1 < GAT_NCHUNK:
            if c >= 1:
                wcp[c - 1].wait()
            issue(c + 1)
        gcp[c].wait()
        wcp[c] = pltpu.async_copy(
            bufs[c % 2],
            xs_hbm.at[pl.ds(base + c * GAT_CHUNK, GAT_CHUNK)],
            wsem[c % 2])
    wcp[GAT_NCHUNK - 2].wait()
    wcp[GAT_NCHUNK - 1].wait()


def _gmm_body(be_ref, bv_ref, x_ref, w1_ref, w2_ref, out_ref, acc_ref):
    c = pl.program_id(0)
    b = pl.program_id(1)

    @pl.when(bv_ref[b] == 1)
    def _():
        xb = x_ref[...].astype(jnp.bfloat16)
        w1b = w1_ref[0].astype(jnp.bfloat16)
        h = jnp.dot(xb, w1b, preferred_element_type=jnp.float32)
        hb = jax.nn.gelu(h).astype(jnp.bfloat16)
        w2b = w2_ref[0].astype(jnp.bfloat16)
        y = jnp.dot(hb, w2b, preferred_element_type=jnp.float32)

        @pl.when(c == 0)
        def _():
            acc_ref[b] = y.astype(jnp.bfloat16)

        @pl.when(c == 1)
        def _():
            out_ref[...] = acc_ref[b].astype(jnp.float32) + y


def _gmm(be, bv, xs, W1, W2):
    grid_spec = pltpu.PrefetchScalarGridSpec(
        num_scalar_prefetch=2,
        grid=(FF_STEPS, NBLK),
        in_specs=[
            pl.BlockSpec((BLK, D_MODEL),
                         lambda c, b, be, bv: (jnp.where(bv[b] == 1, b, 0), 0)),
            pl.BlockSpec((1, D_MODEL, FF_CHUNK),
                         lambda c, b, be, bv:
                         (be[b], 0, jnp.where(bv[b] == 1, c, 0))),
            pl.BlockSpec((1, FF_CHUNK, D_MODEL),
                         lambda c, b, be, bv:
                         (be[b], jnp.where(bv[b] == 1, c, 0), 0)),
        ],
        out_specs=pl.BlockSpec(
            (BLK, D_MODEL),
            lambda c, b, be, bv: (jnp.where(c == 0, 0, b), 0)),
        scratch_shapes=[pltpu.VMEM((NBLK, BLK, D_MODEL), jnp.bfloat16)],
    )
    return pl.pallas_call(
        _gmm_body,
        grid_spec=grid_spec,
        out_shape=jax.ShapeDtypeStruct((NSLOT, D_MODEL), jnp.float32),
        compiler_params=pltpu.CompilerParams(
            vmem_limit_bytes=63 * 1024 * 1024),
    )(be, bv, xs, W1, W2)


CMB_PER_W = N_TOKENS // NWORKER
CMB_CHUNK = 16


CMB_NCHUNK = CMB_PER_W // CMB_CHUNK


@functools.partial(
    pl.kernel, mesh=_MESH,
    compiler_params=pltpu.CompilerParams(needs_layout_passes=False),
    out_type=jax.ShapeDtypeStruct((N_TOKENS, D_MODEL), jnp.float32),
    scratch_types=(
        [pltpu.VMEM((CMB_PER_W,), jnp.int32)] * 2
        + [pltpu.VMEM((CMB_PER_W,), jnp.float32)] * 2
        + [pltpu.VMEM((CMB_CHUNK, D_MODEL), jnp.float32)] * 6
        + [pltpu.SemaphoreType.DMA] * 6
    ),
)
def _combine(y_hbm, pos0_hbm, pos1_hbm, w0_hbm, w1_hbm, out_hbm,
             p0_v, p1_v, w0_v, w1_v, *bufs_sems):
    r0b = bufs_sems[0:2]
    r1b = bufs_sems[2:4]
    obb = bufs_sems[4:6]
    gs0 = bufs_sems[6:8]
    gs1 = bufs_sems[8:10]
    ws = bufs_sems[10:12]
    wid = lax.axis_index("s") * 2 + lax.axis_index("c")
    tbase = wid * CMB_PER_W
    pltpu.sync_copy(pos0_hbm.at[pl.ds(tbase, CMB_PER_W)], p0_v)
    pltpu.sync_copy(pos1_hbm.at[pl.ds(tbase, CMB_PER_W)], p1_v)
    pltpu.sync_copy(w0_hbm.at[pl.ds(tbase, CMB_PER_W)], w0_v)
    pltpu.sync_copy(w1_hbm.at[pl.ds(tbase, CMB_PER_W)], w1_v)

    g0cp = [None] * CMB_NCHUNK
    g1cp = [None] * CMB_NCHUNK
    wcp = [None] * CMB_NCHUNK

    def issue(c):
        g0cp[c] = pltpu.async_copy(
            y_hbm.at[p0_v.at[pl.ds(c * CMB_CHUNK, CMB_CHUNK)]],
            r0b[c % 2], gs0[c % 2])
        g1cp[c] = pltpu.async_copy(
            y_hbm.at[p1_v.at[pl.ds(c * CMB_CHUNK, CMB_CHUNK)]],
            r1b[c % 2], gs1[c % 2])

    issue(0)
    for c in range(CMB_NCHUNK):
        if c + 1 < CMB_NCHUNK:
            issue(c + 1)
        g0cp[c].wait()
        g1cp[c].wait()
        if c >= 2:
            wcp[c - 2].wait()
        wa = w0_v[pl.ds(c * CMB_CHUNK, CMB_CHUNK)]
        wb = w1_v[pl.ds(c * CMB_CHUNK, CMB_CHUNK)]
        r0c = r0b[c % 2]
        r1c = r1b[c % 2]
        obc = obb[c % 2]
        for j in range(CMB_CHUNK):
            s0 = _take(wa, j)
            s1 = _take(wb, j)

            def vbody(v, carry, j=j, s0=s0, s1=s1, r0c=r0c, r1c=r1c, obc=obc):
                for u in range(4):
                    off = v * 64 + u * 16
                    obc[j, pl.ds(off, 16)] = (
                        s0 * r0c[j, pl.ds(off, 16)]
                        + s1 * r1c[j, pl.ds(off, 16)])
                return carry

            lax.fori_loop(0, D_MODEL // 64, vbody, 0)
        wcp[c] = pltpu.async_copy(
            obc, out_hbm.at[pl.ds(tbase + c * CMB_CHUNK, CMB_CHUNK)],
            ws[c % 2])
    wcp[CMB_NCHUNK - 2].wait()
    wcp[CMB_NCHUNK - 1].wait()


def kernel(inputs, W_gate, W1, W2):
    id1, id2, w1, w2 = _gate(inputs, W_gate)
    ids_flat = jnp.concatenate([id1, id2], axis=1).reshape(NPAIR)
    stok, pos0, pos1, be, bv = _route(ids_flat)
    xs = _gather(inputs, stok)
    ys = _gmm(be, bv, xs, W1, W2)
    return _combine(ys, pos0, pos1, w1.reshape(-1), w2.reshape(-1))


# speedup vs baseline: 1.2230x; 1.0617x over previous
import functools

import jax
import jax.numpy as jnp
from jax import lax
from jax.experimental import pallas as pl
from jax.experimental.pallas import tpu as pltpu
from jax.experimental.pallas import tpu_sc as plsc

NUM_EXPERTS = 8
TOP_K = 2
D_MODEL = 1024
D_FF = 4096
N_TOKENS = 2048

NPAIR = N_TOKENS * TOP_K
BLK = 256
NBLK = 24
NSLOT = NBLK * BLK
NWORKER = 32
FF_CHUNK = 2048
FF_STEPS = D_FF // FF_CHUNK

_MESH = plsc.VectorSubcoreMesh(core_axis_name="c", subcore_axis_name="s")


def _gate_body(x_ref, wg_ref, id1_ref, id2_ref, w1_ref, w2_ref, xpk_ref):
    x = x_ref[...]
    wg = wg_ref[...]
    a16 = x[:, :D_MODEL // 2].astype(jnp.bfloat16).astype(jnp.float32)
    b16 = x[:, D_MODEL // 2:].astype(jnp.bfloat16).astype(jnp.float32)
    ua = lax.bitcast_convert_type(a16, jnp.uint32) & jnp.uint32(0xFFFF0000)
    ub = lax.bitcast_convert_type(b16, jnp.uint32) >> 16
    xpk_ref[...] = lax.bitcast_convert_type(ua | ub, jnp.int32)
    logits = jnp.dot(x, wg, preferred_element_type=jnp.float32)
    iota = jax.lax.broadcasted_iota(jnp.int32, logits.shape, 1)
    m1 = jnp.max(logits, axis=1, keepdims=True)
    am1 = jnp.min(jnp.where(logits == m1, iota, NUM_EXPERTS), axis=1,
                  keepdims=True)
    masked = jnp.where(iota == am1, -jnp.inf, logits)
    m2 = jnp.max(masked, axis=1, keepdims=True)
    am2 = jnp.min(jnp.where(masked == m2, iota, NUM_EXPERTS), axis=1,
                  keepdims=True)
    z = jnp.exp(m2 - m1)
    id1_ref[...] = am1
    id2_ref[...] = am2
    w1_ref[...] = 1.0 / (1.0 + z)
    w2_ref[...] = z / (1.0 + z)


def _gate(inputs, W_gate):
    return pl.pallas_call(
        _gate_body,
        out_shape=(
            jax.ShapeDtypeStruct((N_TOKENS, 1), jnp.int32),
            jax.ShapeDtypeStruct((N_TOKENS, 1), jnp.int32),
            jax.ShapeDtypeStruct((N_TOKENS, 1), jnp.float32),
            jax.ShapeDtypeStruct((N_TOKENS, 1), jnp.float32),
            jax.ShapeDtypeStruct((N_TOKENS, D_MODEL // 2), jnp.int32),
        ),
    )(inputs, W_gate)


def _take(vec, idx):
    idxv = jnp.full((16, 1), idx, jnp.int32)
    dnums = lax.GatherDimensionNumbers(
        offset_dims=(), collapsed_slice_dims=(0,), start_index_map=(0,))
    return lax.gather(vec, idxv, dnums, slice_sizes=(1,),
                      mode=lax.GatherScatterMode.PROMISE_IN_BOUNDS)


@functools.partial(
    pl.kernel, mesh=_MESH,
    compiler_params=pltpu.CompilerParams(needs_layout_passes=False),
    out_type=(
        jax.ShapeDtypeStruct((NSLOT,), jnp.int32),
        jax.ShapeDtypeStruct((N_TOKENS,), jnp.int32),
        jax.ShapeDtypeStruct((N_TOKENS,), jnp.int32),
        jax.ShapeDtypeStruct((NWORKER,), jnp.int32),
        jax.ShapeDtypeStruct((NWORKER,), jnp.int32),
    ),
    scratch_types=[
        pltpu.VMEM((NPAIR,), jnp.int32),
        pltpu.VMEM((NSLOT,), jnp.int32),
        pltpu.VMEM((N_TOKENS,), jnp.int32),
        pltpu.VMEM((N_TOKENS,), jnp.int32),
        pltpu.VMEM((16,), jnp.int32),
        pltpu.VMEM((NWORKER,), jnp.int32),
        pltpu.VMEM((NWORKER,), jnp.int32),
    ],
)
def _route(ids_hbm, stok_hbm, pos0_hbm, pos1_hbm, be_hbm, bv_hbm,
           ids_v, stok_v, pos0_v, pos1_v, counts_v, be_v, bv_v):
    cid = lax.axis_index("c")
    sid = lax.axis_index("s")

    @pl.when(jnp.logical_and(cid == 0, sid == 0))
    def _():
        lanes = lax.iota(jnp.int32, 16)
        pltpu.sync_copy(ids_hbm, ids_v)
        counts_v[...] = jnp.zeros((16,), jnp.int32)

        def hist_body(i, carry):
            ids = ids_v[pl.ds(i * 16, 16)]
            plsc.addupdate_scatter(counts_v, [ids],
                                   jnp.ones((16,), jnp.int32))
            return carry

        lax.fori_loop(0, NPAIR // 16, hist_body, 0)

        counts = counts_v[...]
        aligned = ((counts + (BLK - 1)) // BLK) * BLK
        incl = plsc.cumsum(aligned)
        excl = incl - aligned
        total_blk = _take(incl, NUM_EXPERTS - 1) // BLK
        last_e = jnp.max(jnp.where(counts > 0, lanes, 0))

        for j in range(NWORKER // 16):
            bidx = lanes + 16 * j
            acc = jnp.zeros((16,), jnp.int32)
            for e in range(NUM_EXPERTS):
                acc += (bidx * BLK >= _take(excl, e)).astype(jnp.int32)
            eb = jnp.minimum(acc - 1, NUM_EXPERTS - 1)
            valid = (bidx < total_blk).astype(jnp.int32)
            eb = jnp.where(valid == 1, eb, last_e)
            be_v[pl.ds(16 * j, 16)] = eb
            bv_v[pl.ds(16 * j, 16)] = valid

        def zero_body(i, carry):
            stok_v[pl.ds(i * 16, 16)] = jnp.zeros((16,), jnp.int32)
            return carry

        lax.fori_loop(0, NSLOT // 16, zero_body, 0)

        def pos_body(i, counters):
            ids = ids_v[pl.ds(i * 16, 16)]
            p = jnp.zeros((16,), jnp.int32)
            upd = jnp.zeros((16,), jnp.int32)
            for e in range(NUM_EXPERTS):
                m = ids == e
                csum = plsc.cumsum(m.astype(jnp.int32))
                p = jnp.where(m, _take(counters, e) + csum - 1, p)
                upd += jnp.where(lanes == e, _take(csum, 15), 0)
            pairidx = lanes + i * 16
            tok = pairidx // 2
            k = pairidx % 2
            plsc.store_scatter(stok_v, [p], tok)
            plsc.store_scatter(pos0_v, [tok], p, mask=(k == 0))
            plsc.store_scatter(pos1_v, [tok], p, mask=(k == 1))
            return counters + upd

        lax.fori_loop(0, NPAIR // 16, pos_body, excl)

        pltpu.sync_copy(stok_v, stok_hbm)
        pltpu.sync_copy(pos0_v, pos0_hbm)
        pltpu.sync_copy(pos1_v, pos1_hbm)
        pltpu.sync_copy(be_v, be_hbm)
        pltpu.sync_copy(bv_v, bv_hbm)


GAT_PER_W = NSLOT // NWORKER
GAT_CHUNK = 48
GAT_NCHUNK = GAT_PER_W // GAT_CHUNK


@functools.partial(
    pl.kernel, mesh=_MESH,
    compiler_params=pltpu.CompilerParams(needs_layout_passes=False),
    out_type=jax.ShapeDtypeStruct((NSLOT, D_MODEL // 2), jnp.int32),
    scratch_types=(
        [pltpu.VMEM((GAT_PER_W,), jnp.int32)]
        + [pltpu.VMEM((GAT_CHUNK, D_MODEL // 2), jnp.int32)] * GAT_NCHUNK
        + [pltpu.SemaphoreType.DMA] * (2 * GAT_NCHUNK)
    ),
)
def _gather(x_hbm, stok_hbm, xs_hbm, idx_v, *bufs_sems):
    bufs = bufs_sems[0:GAT_NCHUNK]
    gsem = bufs_sems[GAT_NCHUNK:2 * GAT_NCHUNK]
    wsem = bufs_sems[2 * GAT_NCHUNK:3 * GAT_NCHUNK]
    wid = lax.axis_index("s") * 2 + lax.axis_index("c")
    base = wid * GAT_PER_W
    pltpu.sync_copy(stok_hbm.at[pl.ds(base, GAT_PER_W)], idx_v)
    gcp = []
    for c in range(GAT_NCHUNK):
        gcp.append(pltpu.async_copy(
            x_hbm.at[idx_v.at[pl.ds(c * GAT_CHUNK, GAT_CHUNK)]],
            bufs[c], gsem[c]))
    wcp = []
    for c in range(GAT_NCHUNK):
        gcp[c].wait()
        wcp.append(pltpu.async_copy(
            bufs[c],
            xs_hbm.at[pl.ds(base + c * GAT_CHUNK, GAT_CHUNK)],
            wsem[c]))
    for c in range(GAT_NCHUNK):
        wcp[c].wait()


def _gmm_body(be_ref, bv_ref, x_ref, w1_ref, w2_ref, out_ref, acc_ref):
    c = pl.program_id(0)
    b = pl.program_id(1)

    @pl.when(bv_ref[b] == 1)
    def _():
        u = lax.bitcast_convert_type(x_ref[...], jnp.uint32)
        xa = lax.bitcast_convert_type(
            u & jnp.uint32(0xFFFF0000), jnp.float32).astype(jnp.bfloat16)
        xc = lax.bitcast_convert_type(
            u << 16, jnp.float32).astype(jnp.bfloat16)
        xb = jnp.concatenate([xa, xc], axis=1)
        w1b = w1_ref[0].astype(jnp.bfloat16)
        h = jnp.dot(xb, w1b, preferred_element_type=jnp.float32)
        hb = jax.nn.gelu(h).astype(jnp.bfloat16)
        w2b = w2_ref[0].astype(jnp.bfloat16)
        y = jnp.dot(hb, w2b, preferred_element_type=jnp.float32)

        @pl.when(c == 0)
        def _():
            acc_ref[b] = y.astype(jnp.bfloat16)

        @pl.when(c == 1)
        def _():
            out_ref[...] = acc_ref[b].astype(jnp.float32) + y


def _gmm(be, bv, xs, W1, W2):
    grid_spec = pltpu.PrefetchScalarGridSpec(
        num_scalar_prefetch=2,
        grid=(FF_STEPS, NBLK),
        in_specs=[
            pl.BlockSpec((BLK, D_MODEL // 2),
                         lambda c, b, be, bv: (jnp.where(bv[b] == 1, b, 0), 0)),
            pl.BlockSpec((1, D_MODEL, FF_CHUNK),
                         lambda c, b, be, bv:
                         (be[b], 0, jnp.where(bv[b] == 1, c, 0))),
            pl.BlockSpec((1, FF_CHUNK, D_MODEL),
                         lambda c, b, be, bv:
                         (be[b], jnp.where(bv[b] == 1, c, 0), 0)),
        ],
        out_specs=pl.BlockSpec(
            (BLK, D_MODEL),
            lambda c, b, be, bv: (jnp.where(c == 0, 0, b), 0)),
        scratch_shapes=[pltpu.VMEM((NBLK, BLK, D_MODEL), jnp.bfloat16)],
    )
    return pl.pallas_call(
        _gmm_body,
        grid_spec=grid_spec,
        out_shape=jax.ShapeDtypeStruct((NSLOT, D_MODEL), jnp.float32),
        compiler_params=pltpu.CompilerParams(
            vmem_limit_bytes=63 * 1024 * 1024),
    )(be, bv, xs, W1, W2)


CMB_PER_W = N_TOKENS // NWORKER
CMB_CHUNK = 16


CMB_NCHUNK = CMB_PER_W // CMB_CHUNK


@functools.partial(
    pl.kernel, mesh=_MESH,
    compiler_params=pltpu.CompilerParams(needs_layout_passes=False),
    out_type=jax.ShapeDtypeStruct((N_TOKENS, D_MODEL), jnp.float32),
    scratch_types=(
        [pltpu.VMEM((CMB_PER_W,), jnp.int32)] * 2
        + [pltpu.VMEM((CMB_PER_W,), jnp.float32)] * 2
        + [pltpu.VMEM((CMB_CHUNK, D_MODEL), jnp.float32)] * 6
        + [pltpu.SemaphoreType.DMA] * 6
    ),
)
def _combine(y_hbm, pos0_hbm, pos1_hbm, w0_hbm, w1_hbm, out_hbm,
             p0_v, p1_v, w0_v, w1_v, *bufs_sems):
    r0b = bufs_sems[0:2]
    r1b = bufs_sems[2:4]
    obb = bufs_sems[4:6]
    gs0 = bufs_sems[6:8]
    gs1 = bufs_sems[8:10]
    ws = bufs_sems[10:12]
    wid = lax.axis_index("s") * 2 + lax.axis_index("c")
    tbase = wid * CMB_PER_W
    pltpu.sync_copy(pos0_hbm.at[pl.ds(tbase, CMB_PER_W)], p0_v)
    pltpu.sync_copy(pos1_hbm.at[pl.ds(tbase, CMB_PER_W)], p1_v)
    pltpu.sync_copy(w0_hbm.at[pl.ds(tbase, CMB_PER_W)], w0_v)
    pltpu.sync_copy(w1_hbm.at[pl.ds(tbase, CMB_PER_W)], w1_v)

    g0cp = [None] * CMB_NCHUNK
    g1cp = [None] * CMB_NCHUNK
    wcp = [None] * CMB_NCHUNK

    def issue(c):
        g0cp[c] = pltpu.async_copy(
            y_hbm.at[p0_v.at[pl.ds(c * CMB_CHUNK, CMB_CHUNK)]],
            r0b[c % 2], gs0[c % 2])
        g1cp[c] = pltpu.async_copy(
            y_hbm.at[p1_v.at[pl.ds(c * CMB_CHUNK, CMB_CHUNK)]],
            r1b[c % 2], gs1[c % 2])

    issue(0)
    for c in range(CMB_NCHUNK):
        if c + 1 < CMB_NCHUNK:
            issue(c + 1)
        g0cp[c].wait()
        g1cp[c].wait()
        if c >= 2:
            wcp[c - 2].wait()
        wa = w0_v[pl.ds(c * CMB_CHUNK, CMB_CHUNK)]
        wb = w1_v[pl.ds(c * CMB_CHUNK, CMB_CHUNK)]
        r0c = r0b[c % 2]
        r1c = r1b[c % 2]
        obc = obb[c % 2]
        for j in range(CMB_CHUNK):
            s0 = _take(wa, j)
            s1 = _take(wb, j)

            def vbody(v, carry, j=j, s0=s0, s1=s1, r0c=r0c, r1c=r1c, obc=obc):
                for u in range(4):
                    off = v * 64 + u * 16
                    obc[j, pl.ds(off, 16)] = (
                        s0 * r0c[j, pl.ds(off, 16)]
                        + s1 * r1c[j, pl.ds(off, 16)])
                return carry

            lax.fori_loop(0, D_MODEL // 64, vbody, 0)
        wcp[c] = pltpu.async_copy(
            obc, out_hbm.at[pl.ds(tbase + c * CMB_CHUNK, CMB_CHUNK)],
            ws[c % 2])
    wcp[CMB_NCHUNK - 2].wait()
    wcp[CMB_NCHUNK - 1].wait()


def kernel(inputs, W_gate, W1, W2):
    id1, id2, w1, w2, x_pk = _gate(inputs, W_gate)
    ids_flat = jnp.concatenate([id1, id2], axis=1).reshape(NPAIR)
    stok, pos0, pos1, be, bv = _route(ids_flat)
    xs = _gather(x_pk, stok)
    ys = _gmm(be, bv, xs, W1, W2)
    return _combine(ys, pos0, pos1, w1.reshape(-1), w2.reshape(-1))


# speedup vs baseline: 1.5450x; 1.2633x over previous
import functools

import jax
import jax.numpy as jnp
from jax import lax
from jax.experimental import pallas as pl
from jax.experimental.pallas import tpu as pltpu
from jax.experimental.pallas import tpu_sc as plsc

NUM_EXPERTS = 8
TOP_K = 2
D_MODEL = 1024
D_FF = 4096
N_TOKENS = 2048

NPAIR = N_TOKENS * TOP_K
BLK = 256
NBLK = 24
NSLOT = NBLK * BLK
NWORKER = 32
FF_CHUNK = 2048
FF_STEPS = D_FF // FF_CHUNK

_MESH = plsc.VectorSubcoreMesh(core_axis_name="c", subcore_axis_name="s")


def _gate_body(x_ref, wg_ref, id1_ref, id2_ref, w1_ref, w2_ref, xpk_ref):
    x = x_ref[...]
    wg = wg_ref[...]
    xpk_ref[...] = x.astype(jnp.bfloat16)
    logits = jnp.dot(x, wg, preferred_element_type=jnp.float32)
    iota = jax.lax.broadcasted_iota(jnp.int32, logits.shape, 1)
    m1 = jnp.max(logits, axis=1, keepdims=True)
    am1 = jnp.min(jnp.where(logits == m1, iota, NUM_EXPERTS), axis=1,
                  keepdims=True)
    masked = jnp.where(iota == am1, -jnp.inf, logits)
    m2 = jnp.max(masked, axis=1, keepdims=True)
    am2 = jnp.min(jnp.where(masked == m2, iota, NUM_EXPERTS), axis=1,
                  keepdims=True)
    z = jnp.exp(m2 - m1)
    id1_ref[...] = am1
    id2_ref[...] = am2
    w1_ref[...] = 1.0 / (1.0 + z)
    w2_ref[...] = z / (1.0 + z)


def _gate(inputs, W_gate):
    return pl.pallas_call(
        _gate_body,
        out_shape=(
            jax.ShapeDtypeStruct((N_TOKENS, 1), jnp.int32),
            jax.ShapeDtypeStruct((N_TOKENS, 1), jnp.int32),
            jax.ShapeDtypeStruct((N_TOKENS, 1), jnp.float32),
            jax.ShapeDtypeStruct((N_TOKENS, 1), jnp.float32),
            jax.ShapeDtypeStruct((N_TOKENS, D_MODEL), jnp.bfloat16),
        ),
    )(inputs, W_gate)


def _take(vec, idx):
    idxv = jnp.full((16, 1), idx, jnp.int32)
    dnums = lax.GatherDimensionNumbers(
        offset_dims=(), collapsed_slice_dims=(0,), start_index_map=(0,))
    return lax.gather(vec, idxv, dnums, slice_sizes=(1,),
                      mode=lax.GatherScatterMode.PROMISE_IN_BOUNDS)


@functools.partial(
    pl.kernel, mesh=_MESH,
    compiler_params=pltpu.CompilerParams(needs_layout_passes=False),
    out_type=(
        jax.ShapeDtypeStruct((NSLOT,), jnp.int32),
        jax.ShapeDtypeStruct((N_TOKENS,), jnp.int32),
        jax.ShapeDtypeStruct((N_TOKENS,), jnp.int32),
        jax.ShapeDtypeStruct((NWORKER,), jnp.int32),
        jax.ShapeDtypeStruct((NWORKER,), jnp.int32),
    ),
    scratch_types=[
        pltpu.VMEM((NPAIR,), jnp.int32),
        pltpu.VMEM((NSLOT,), jnp.int32),
        pltpu.VMEM((N_TOKENS,), jnp.int32),
        pltpu.VMEM((N_TOKENS,), jnp.int32),
        pltpu.VMEM((16,), jnp.int32),
        pltpu.VMEM((NWORKER,), jnp.int32),
        pltpu.VMEM((NWORKER,), jnp.int32),
    ],
)
def _route(ids_hbm, stok_hbm, pos0_hbm, pos1_hbm, be_hbm, bv_hbm,
           ids_v, stok_v, pos0_v, pos1_v, counts_v, be_v, bv_v):
    cid = lax.axis_index("c")
    sid = lax.axis_index("s")

    @pl.when(jnp.logical_and(cid == 0, sid == 0))
    def _():
        lanes = lax.iota(jnp.int32, 16)
        pltpu.sync_copy(ids_hbm, ids_v)
        counts_v[...] = jnp.zeros((16,), jnp.int32)

        def hist_body(i, carry):
            ids = ids_v[pl.ds(i * 16, 16)]
            plsc.addupdate_scatter(counts_v, [ids],
                                   jnp.ones((16,), jnp.int32))
            return carry

        lax.fori_loop(0, NPAIR // 16, hist_body, 0)

        counts = counts_v[...]
        aligned = ((counts + (BLK - 1)) // BLK) * BLK
        incl = plsc.cumsum(aligned)
        excl = incl - aligned
        total_blk = _take(incl, NUM_EXPERTS - 1) // BLK
        last_e = jnp.max(jnp.where(counts > 0, lanes, 0))

        for j in range(NWORKER // 16):
            bidx = lanes + 16 * j
            acc = jnp.zeros((16,), jnp.int32)
            for e in range(NUM_EXPERTS):
                acc += (bidx * BLK >= _take(excl, e)).astype(jnp.int32)
            eb = jnp.minimum(acc - 1, NUM_EXPERTS - 1)
            valid = (bidx < total_blk).astype(jnp.int32)
            eb = jnp.where(valid == 1, eb, last_e)
            be_v[pl.ds(16 * j, 16)] = eb
            bv_v[pl.ds(16 * j, 16)] = valid

        def zero_body(i, carry):
            stok_v[pl.ds(i * 16, 16)] = jnp.zeros((16,), jnp.int32)
            return carry

        lax.fori_loop(0, NSLOT // 16, zero_body, 0)

        def pos_body(i, counters):
            ids = ids_v[pl.ds(i * 16, 16)]
            p = jnp.zeros((16,), jnp.int32)
            upd = jnp.zeros((16,), jnp.int32)
            for e in range(NUM_EXPERTS):
                m = ids == e
                csum = plsc.cumsum(m.astype(jnp.int32))
                p = jnp.where(m, _take(counters, e) + csum - 1, p)
                upd += jnp.where(lanes == e, _take(csum, 15), 0)
            pairidx = lanes + i * 16
            tok = pairidx // 2
            k = pairidx % 2
            plsc.store_scatter(stok_v, [p], tok)
            plsc.store_scatter(pos0_v, [tok], p, mask=(k == 0))
            plsc.store_scatter(pos1_v, [tok], p, mask=(k == 1))
            return counters + upd

        lax.fori_loop(0, NPAIR // 16, pos_body, excl)

        pltpu.sync_copy(stok_v, stok_hbm)
        pltpu.sync_copy(pos0_v, pos0_hbm)
        pltpu.sync_copy(pos1_v, pos1_hbm)
        pltpu.sync_copy(be_v, be_hbm)
        pltpu.sync_copy(bv_v, bv_hbm)


def _gmm_body(be_ref, bv_ref, stok_ref, x_ref, w1_ref, w2_ref, out_ref,
              acc_ref):
    c = pl.program_id(0)
    b = pl.program_id(1)

    @pl.when(bv_ref[b] == 1)
    def _():
        tok = stok_ref[...]
        ti = lax.broadcasted_iota(jnp.int32, (BLK, N_TOKENS), 1)
        sel = (ti == tok).astype(jnp.bfloat16)
        xb = jnp.dot(sel, x_ref[...],
                     preferred_element_type=jnp.float32).astype(jnp.bfloat16)
        w1b = w1_ref[0].astype(jnp.bfloat16)
        h = jnp.dot(xb, w1b, preferred_element_type=jnp.float32)
        hb = jax.nn.gelu(h).astype(jnp.bfloat16)
        w2b = w2_ref[0].astype(jnp.bfloat16)
        y = jnp.dot(hb, w2b, preferred_element_type=jnp.float32)

        @pl.when(c == 0)
        def _():
            acc_ref[b] = y.astype(jnp.bfloat16)

        @pl.when(c == 1)
        def _():
            out_ref[...] = acc_ref[b].astype(jnp.float32) + y


def _gmm(be, bv, stok, x_bf, W1, W2):
    grid_spec = pltpu.PrefetchScalarGridSpec(
        num_scalar_prefetch=2,
        grid=(FF_STEPS, NBLK),
        in_specs=[
            pl.BlockSpec((BLK, 1), lambda c, b, be, bv: (b, 0)),
            pl.BlockSpec((N_TOKENS, D_MODEL),
                         lambda c, b, be, bv: (0, 0)),
            pl.BlockSpec((1, D_MODEL, FF_CHUNK),
                         lambda c, b, be, bv:
                         (be[b], 0, jnp.where(bv[b] == 1, c, 0))),
            pl.BlockSpec((1, FF_CHUNK, D_MODEL),
                         lambda c, b, be, bv:
                         (be[b], jnp.where(bv[b] == 1, c, 0), 0)),
        ],
        out_specs=pl.BlockSpec(
            (BLK, D_MODEL),
            lambda c, b, be, bv: (jnp.where(c == 0, 0, b), 0)),
        scratch_shapes=[pltpu.VMEM((NBLK, BLK, D_MODEL), jnp.bfloat16)],
    )
    return pl.pallas_call(
        _gmm_body,
        grid_spec=grid_spec,
        out_shape=jax.ShapeDtypeStruct((NSLOT, D_MODEL), jnp.float32),
        compiler_params=pltpu.CompilerParams(
            vmem_limit_bytes=63 * 1024 * 1024),
    )(be, bv, stok.reshape(NSLOT, 1), x_bf, W1, W2)


CMB_PER_W = N_TOKENS // NWORKER
CMB_CHUNK = 16


CMB_NCHUNK = CMB_PER_W // CMB_CHUNK


@functools.partial(
    pl.kernel, mesh=_MESH,
    compiler_params=pltpu.CompilerParams(needs_layout_passes=False),
    out_type=jax.ShapeDtypeStruct((N_TOKENS, D_MODEL), jnp.float32),
    scratch_types=(
        [pltpu.VMEM((CMB_PER_W,), jnp.int32)] * 2
        + [pltpu.VMEM((CMB_PER_W,), jnp.float32)] * 2
        + [pltpu.VMEM((CMB_CHUNK, D_MODEL), jnp.float32)] * 6
        + [pltpu.SemaphoreType.DMA] * 6
    ),
)
def _combine(y_hbm, pos0_hbm, pos1_hbm, w0_hbm, w1_hbm, out_hbm,
             p0_v, p1_v, w0_v, w1_v, *bufs_sems):
    r0b = bufs_sems[0:2]
    r1b = bufs_sems[2:4]
    obb = bufs_sems[4:6]
    gs0 = bufs_sems[6:8]
    gs1 = bufs_sems[8:10]
    ws = bufs_sems[10:12]
    wid = lax.axis_index("s") * 2 + lax.axis_index("c")
    tbase = wid * CMB_PER_W
    pltpu.sync_copy(pos0_hbm.at[pl.ds(tbase, CMB_PER_W)], p0_v)
    pltpu.sync_copy(pos1_hbm.at[pl.ds(tbase, CMB_PER_W)], p1_v)
    pltpu.sync_copy(w0_hbm.at[pl.ds(tbase, CMB_PER_W)], w0_v)
    pltpu.sync_copy(w1_hbm.at[pl.ds(tbase, CMB_PER_W)], w1_v)

    g0cp = [None] * CMB_NCHUNK
    g1cp = [None] * CMB_NCHUNK
    wcp = [None] * CMB_NCHUNK

    def issue(c):
        g0cp[c] = pltpu.async_copy(
            y_hbm.at[p0_v.at[pl.ds(c * CMB_CHUNK, CMB_CHUNK)]],
            r0b[c % 2], gs0[c % 2])
        g1cp[c] = pltpu.async_copy(
            y_hbm.at[p1_v.at[pl.ds(c * CMB_CHUNK, CMB_CHUNK)]],
            r1b[c % 2], gs1[c % 2])

    issue(0)
    for c in range(CMB_NCHUNK):
        if c + 1 < CMB_NCHUNK:
            issue(c + 1)
        g0cp[c].wait()
        g1cp[c].wait()
        if c >= 2:
            wcp[c - 2].wait()
        wa = w0_v[pl.ds(c * CMB_CHUNK, CMB_CHUNK)]
        wb = w1_v[pl.ds(c * CMB_CHUNK, CMB_CHUNK)]
        r0c = r0b[c % 2]
        r1c = r1b[c % 2]
        obc = obb[c % 2]
        for j in range(CMB_CHUNK):
            s0 = _take(wa, j)
            s1 = _take(wb, j)

            def vbody(v, carry, j=j, s0=s0, s1=s1, r0c=r0c, r1c=r1c, obc=obc):
                for u in range(4):
                    off = v * 64 + u * 16
                    obc[j, pl.ds(off, 16)] = (
                        s0 * r0c[j, pl.ds(off, 16)]
                        + s1 * r1c[j, pl.ds(off, 16)])
                return carry

            lax.fori_loop(0, D_MODEL // 64, vbody, 0)
        wcp[c] = pltpu.async_copy(
            obc, out_hbm.at[pl.ds(tbase + c * CMB_CHUNK, CMB_CHUNK)],
            ws[c % 2])
    wcp[CMB_NCHUNK - 2].wait()
    wcp[CMB_NCHUNK - 1].wait()


def kernel(inputs, W_gate, W1, W2):
    id1, id2, w1, w2, x_bf = _gate(inputs, W_gate)
    ids_flat = jnp.concatenate([id1, id2], axis=1).reshape(NPAIR)
    stok, pos0, pos1, be, bv = _route(ids_flat)
    ys = _gmm(be, bv, stok, x_bf, W1, W2)
    return _combine(ys, pos0, pos1, w1.reshape(-1), w2.reshape(-1))
